# Initial kernel scaffold; baseline (speedup 1.0000x reference)
import jax
import jax.numpy as jnp
from jax.experimental import pallas as pl


def kernel(x, edge_index, edge_attr, batch, ee_w, ee_b, ne_w, ne_b, m1_w1, m1_b1, m1_w2, m1_b2, m2_w1, m2_b1, m2_w2, m2_b2, m3_w1, m3_b1, m3_w2, m3_b2, bn1_g, bn1_b, bn2_g, bn2_b, bn3_g, bn3_b, fc_w, fc_b):
    raise NotImplementedError("write your pallas kernel here")


# speedup vs baseline: 4.4499x; 4.4499x over previous
import functools

import jax
import jax.numpy as jnp
from jax import lax
from jax.experimental import pallas as pl
from jax.experimental.pallas import tpu as pltpu
from jax.experimental.pallas import tpu_sc as plsc

NN = 10000
NP = 10240
EE = 320000
GG = 64
NC = 2
NS = 16
NW = NC * NS
CB = 128
EPW = -(-EE // (NW * CB)) * CB
NCH = EPW // CB
EP = EPW * NW
CW = 16
RPT = NP // NS


def _edge_body_cnt(u_hbm, v_hbm, dsti_hbm, srci_hbm, acc_out, cnt_out,
                   dst_v, src_v, u_rows, v_rows, ones_v, acc_sh, cnt_sh,
                   sem_u, sem_v, *, H):
    _edge_common(u_hbm, v_hbm, dsti_hbm, srci_hbm, acc_out, cnt_out,
                 dst_v, src_v, u_rows, v_rows, ones_v, acc_sh, cnt_sh,
                 sem_u, sem_v, H=H)


def _edge_body_nocnt(u_hbm, v_hbm, dsti_hbm, srci_hbm, acc_out,
                     dst_v, src_v, u_rows, v_rows, acc_sh,
                     sem_u, sem_v, *, H):
    _edge_common(u_hbm, v_hbm, dsti_hbm, srci_hbm, acc_out, None,
                 dst_v, src_v, u_rows, v_rows, None, acc_sh, None,
                 sem_u, sem_v, H=H)


def _edge_common(u_hbm, v_hbm, dsti_hbm, srci_hbm, acc_out, cnt_out,
                 dst_v, src_v, u_rows, v_rows, ones_v, acc_sh, cnt_sh,
                 sem_u, sem_v, *, H):
    cid = lax.axis_index("c")
    sid = lax.axis_index("s")
    wid = cid * NS + sid

    pltpu.sync_copy(dsti_hbm.at[wid], dst_v)
    pltpu.sync_copy(srci_hbm.at[wid], src_v)

    def zrow(i, c):
        for k in range(H // 16):
            u_rows[i, pl.ds(k * 16, 16)] = jnp.zeros((16,), jnp.float32)
        return c
    lax.fori_loop(0, CB, zrow, 0, unroll=2)
    for j in range(RPT // CB):
        pltpu.sync_copy(u_rows, acc_sh.at[pl.ds(sid * RPT + j * CB, CB)])

    if cnt_out is not None:
        def crow(i, c):
            ones_v[i, pl.ds(0, 16)] = jnp.zeros((16,), jnp.float32)
            return c
        lax.fori_loop(0, CB, crow, 0, unroll=2)
        for j in range(RPT // CB):
            pltpu.sync_copy(ones_v, cnt_sh.at[pl.ds(sid * RPT + j * CB, CB)])

        def orow(i, c):
            ones_v[i, pl.ds(0, 16)] = jnp.full((16,), 1.0, jnp.float32)
            return c
        lax.fori_loop(0, CB, orow, 0, unroll=2)

    plsc.subcore_barrier()

    def chunk(c, carry):
        cp_u = pltpu.async_copy(u_hbm.at[dst_v.at[c]], u_rows, sem_u)
        cp_v = pltpu.async_copy(v_hbm.at[src_v.at[c]], v_rows, sem_v)
        cp_u.wait()
        cp_v.wait()

        def row(i, cc):
            for k in range(H // 16):
                sl = pl.ds(k * 16, 16)
                u_rows[i, sl] = jnp.maximum(u_rows[i, sl] + v_rows[i, sl], 0.0)
            return cc
        lax.fori_loop(0, CB, row, 0, unroll=2)

        pltpu.sync_copy(u_rows, acc_sh.at[dst_v.at[c]], add=True)
        if cnt_out is not None:
            pltpu.sync_copy(ones_v, cnt_sh.at[dst_v.at[c]], add=True)
        return carry
    lax.fori_loop(0, NCH, chunk, 0)

    plsc.subcore_barrier()
    pltpu.sync_copy(acc_sh.at[pl.ds(sid * RPT, RPT)],
                    acc_out.at[cid].at[pl.ds(sid * RPT, RPT)])
    if cnt_out is not None:
        pltpu.sync_copy(cnt_sh.at[pl.ds(sid * RPT, RPT)],
                        cnt_out.at[cid].at[pl.ds(sid * RPT, RPT)])


def _make_edge_kernel(H, with_cnt):
    mesh = plsc.VectorSubcoreMesh(core_axis_name="c", subcore_axis_name="s",
                                  num_cores=NC, num_subcores=NS)
    acc_t = jax.ShapeDtypeStruct((NC, NP, H), jnp.float32)
    cnt_t = jax.ShapeDtypeStruct((NC, NP, CW), jnp.float32)
    out_type = (acc_t, cnt_t) if with_cnt else acc_t
    scratch = [
        pltpu.VMEM((NCH, CB), jnp.int32),
        pltpu.VMEM((NCH, CB), jnp.int32),
        pltpu.VMEM((CB, H), jnp.float32),
        pltpu.VMEM((CB, H), jnp.float32),
    ]
    if with_cnt:
        scratch.append(pltpu.VMEM((CB, CW), jnp.float32))
    scratch.append(pltpu.VMEM_SHARED((NP, H), jnp.float32))
    if with_cnt:
        scratch.append(pltpu.VMEM_SHARED((NP, CW), jnp.float32))
    scratch += [pltpu.SemaphoreType.DMA, pltpu.SemaphoreType.DMA]
    body = _edge_body_cnt if with_cnt else _edge_body_nocnt
    return pl.kernel(functools.partial(body, H=H), out_type=out_type,
                     mesh=mesh, scratch_types=tuple(scratch),
                     compiler_params=pltpu.CompilerParams(
                         use_tc_tiling_on_sc=False))


def _tc0_body(x_ref, new_ref, neb_ref, w1_ref, b1_ref, u_ref, v_ref):
    h = jnp.dot(x_ref[...], new_ref[...],
                preferred_element_type=jnp.float32) + neb_ref[...]
    F = h.shape[1]
    wl = w1_ref[:F, :]
    wr = w1_ref[F:, :]
    u_ref[...] = jnp.dot(h, wl - wr, preferred_element_type=jnp.float32) + b1_ref[...]
    v_ref[...] = jnp.dot(h, wr, preferred_element_type=jnp.float32)


def _tc_mid1_body(acca_ref, accb_ref, cnt_ref, w2_ref, b2_ref, bng_ref,
                  bnb_ref, w1n_ref, b1n_ref, u_ref, v_ref):
    cnt = cnt_ref[0, :, 0:1] + cnt_ref[1, :, 0:1]
    s = jnp.concatenate([acca_ref[0] + acca_ref[1],
                         accb_ref[0] + accb_ref[1]], axis=1)
    mean = s / jnp.maximum(cnt, 1.0)
    _tc_mid_tail(mean, cnt, w2_ref, b2_ref, bng_ref, bnb_ref,
                 w1n_ref, b1n_ref, u_ref, v_ref)


def _tc_mid_body(acc_ref, cnt_ref, w2_ref, b2_ref, bng_ref, bnb_ref,
                 w1n_ref, b1n_ref, u_ref, v_ref):
    cnt = cnt_ref[0, :, 0:1] + cnt_ref[1, :, 0:1]
    mean = (acc_ref[0] + acc_ref[1]) / jnp.maximum(cnt, 1.0)
    _tc_mid_tail(mean, cnt, w2_ref, b2_ref, bng_ref, bnb_ref,
                 w1n_ref, b1n_ref, u_ref, v_ref)


def _tc_mid_tail(mean, cnt, w2_ref, b2_ref, bng_ref, bnb_ref,
                 w1n_ref, b1n_ref, u_ref, v_ref):
    g = jnp.dot(mean, w2_ref[...],
                preferred_element_type=jnp.float32) + b2_ref[...]
    g = jnp.where(cnt > 0.0, g, 0.0)
    rm = (lax.broadcasted_iota(jnp.int32, (NP, 1), 0) < NN).astype(jnp.float32)
    mu = jnp.sum(g * rm, axis=0, keepdims=True) / NN
    d = (g - mu) * rm
    var = jnp.sum(d * d, axis=0, keepdims=True) / NN
    h = jnp.maximum((g - mu) / jnp.sqrt(var + 1e-5) * bng_ref[...] + bnb_ref[...], 0.0)
    h = h * rm
    F = h.shape[1]
    wl = w1n_ref[:F, :]
    wr = w1n_ref[F:, :]
    u_ref[...] = jnp.dot(h, wl - wr, preferred_element_type=jnp.float32) + b1n_ref[...]
    v_ref[...] = jnp.dot(h, wr, preferred_element_type=jnp.float32)


def _tc_fin_body(acc_ref, cnt_ref, w2_ref, b2_ref, bng_ref, bnb_ref,
                 batch_ref, fcw_ref, fcb_ref, out_ref):
    cnt = cnt_ref[0, :, 0:1] + cnt_ref[1, :, 0:1]
    mean = (acc_ref[0] + acc_ref[1]) / jnp.maximum(cnt, 1.0)
    g = jnp.dot(mean, w2_ref[...],
                preferred_element_type=jnp.float32) + b2_ref[...]
    g = jnp.where(cnt > 0.0, g, 0.0)
    rm = (lax.broadcasted_iota(jnp.int32, (NP, 1), 0) < NN).astype(jnp.float32)
    mu = jnp.sum(g * rm, axis=0, keepdims=True) / NN
    d = (g - mu) * rm
    var = jnp.sum(d * d, axis=0, keepdims=True) / NN
    h = jnp.maximum((g - mu) / jnp.sqrt(var + 1e-5) * bng_ref[...] + bnb_ref[...], 0.0)
    h = h * rm
    oh = (batch_ref[...] == lax.broadcasted_iota(jnp.int32, (GG, NP), 0)
          ).astype(jnp.float32)
    gs = jnp.dot(oh, h, preferred_element_type=jnp.float32)
    gc = jnp.sum(oh, axis=1, keepdims=True)
    pooled = gs / jnp.maximum(gc, 1.0)
    o = jnp.dot(pooled, fcw_ref[...],
                preferred_element_type=jnp.float32) + fcb_ref[...]
    out_ref[...] = jax.nn.sigmoid(o)


def _tc0(x_pad, ne_w, ne_b, m1_w1, m1_b1):
    return pl.pallas_call(
        _tc0_body,
        out_shape=(jax.ShapeDtypeStruct((NP, 128), jnp.float32),
                   jax.ShapeDtypeStruct((NP, 128), jnp.float32)),
    )(x_pad, ne_w, ne_b, m1_w1, m1_b1)


def _tc_mid1(acca, accb, cnt, w2, b2, bng, bnb, w1n, b1n, hn):
    return pl.pallas_call(
        _tc_mid1_body,
        out_shape=(jax.ShapeDtypeStruct((NP, hn), jnp.float32),
                   jax.ShapeDtypeStruct((NP, hn), jnp.float32)),
    )(acca, accb, cnt, w2, b2, bng, bnb, w1n, b1n)


def _tc_mid(acc, cnt, w2, b2, bng, bnb, w1n, b1n, hn):
    return pl.pallas_call(
        _tc_mid_body,
        out_shape=(jax.ShapeDtypeStruct((NP, hn), jnp.float32),
                   jax.ShapeDtypeStruct((NP, hn), jnp.float32)),
    )(acc, cnt, w2, b2, bng, bnb, w1n, b1n)


def _tc_fin(acc, cnt, w2, b2, bng, bnb, batch_row, fc_w, fc_b):
    return pl.pallas_call(
        _tc_fin_body,
        out_shape=jax.ShapeDtypeStruct((GG, 5), jnp.float32),
    )(acc, cnt, w2, b2, bng, bnb, batch_row, fc_w, fc_b)


_edge_k1a = _make_edge_kernel(64, with_cnt=True)
_edge_k64 = _make_edge_kernel(64, with_cnt=False)
_edge_k32 = _make_edge_kernel(32, with_cnt=False)


@jax.jit
def kernel(x, edge_index, edge_attr, batch, ee_w, ee_b, ne_w, ne_b,
           m1_w1, m1_b1, m1_w2, m1_b2, m2_w1, m2_b1, m2_w2, m2_b2,
           m3_w1, m3_b1, m3_w2, m3_b2, bn1_g, bn1_b, bn2_g, bn2_b,
           bn3_g, bn3_b, fc_w, fc_b):
    src = edge_index[0]
    dst = edge_index[1]
    pad = jnp.full((EP - EE,), NP - 1, jnp.int32)
    srci = jnp.concatenate([src, pad]).reshape(NW, NCH, CB)
    dsti = jnp.concatenate([dst, pad]).reshape(NW, NCH, CB)
    x_pad = jnp.pad(x, ((0, NP - NN), (0, 0)))
    batch_row = jnp.pad(batch, (0, NP - NN), constant_values=GG).reshape(1, NP)

    r1 = lambda a: a.reshape(1, -1)

    u1, v1 = _tc0(x_pad, ne_w, r1(ne_b), m1_w1, r1(m1_b1))
    acc1a, cnt = _edge_k1a(u1[:, :64], v1[:, :64], dsti, srci)
    acc1b = _edge_k64(u1[:, 64:], v1[:, 64:], dsti, srci)
    u2, v2 = _tc_mid1(acc1a, acc1b, cnt, m1_w2, r1(m1_b2), r1(bn1_g),
                      r1(bn1_b), m2_w1, r1(m2_b1), 64)
    acc2 = _edge_k64(u2, v2, dsti, srci)
    u3, v3 = _tc_mid(acc2, cnt, m2_w2, r1(m2_b2), r1(bn2_g), r1(bn2_b),
                     m3_w1, r1(m3_b1), 32)
    acc3 = _edge_k32(u3, v3, dsti, srci)
    return _tc_fin(acc3, cnt, m3_w2, r1(m3_b2), r1(bn3_g), r1(bn3_b),
                   batch_row, fc_w, r1(fc_b))
